# baseline (speedup 1.0000x reference)
import functools

import jax
import jax.numpy as jnp
from jax import lax
from jax.experimental import pallas as pl
from jax.experimental.pallas import tpu as pltpu
from jax.experimental.pallas import tpu_sc as plsc

N_SEG = 100000
M_ELEM = 6400000
K = 16
NW = 32
SEG_PER_W = 3128
SEG_LAST = N_SEG - (NW - 1) * SEG_PER_W
YLOC = 3136
CHUNK = 16384
NB = 48

def _merge_top16(top, piece):
    ps, _ = plsc.sort_key_val(piece, piece, descending=True)
    m = jnp.maximum(top, jnp.flip(ps, 0))
    ts, _ = plsc.sort_key_val(m, m, descending=True)
    return ts


def _negv():
    return jnp.full((K,), -jnp.inf, dtype=jnp.float32)


def _sort_pool_body(e_hbm, yv_hbm, w_hbm, out_hbm,
                    ids_v, vals_v, yloc, wk_v, tmp_ids, idx_v, probe_v,
                    sem0, sem1):
    wid = lax.axis_index("c") * 16 + lax.axis_index("s")
    pltpu.sync_copy(w_hbm, wk_v)
    iota = lax.iota(jnp.int32, K)
    lane0 = iota == 0

    w_raw = wk_v[...]
    wsrt, _ = plsc.sort_key_val(w_raw, w_raw, descending=True)
    w_exp = jnp.exp(w_raw - wsrt[0])
    wk = w_exp / jnp.sum(w_exp)

    s0 = wid * SEG_PER_W
    s1m1 = jnp.minimum(s0 + SEG_PER_W, N_SEG) - 1

    big = jnp.int32(2 ** 30)
    iota1 = iota + 1
    tgt_a = s0
    tgt_b = s0 + SEG_PER_W

    def bs_body(i, st):
        lo_a, hi_a, lo_b, hi_b = st
        wa = hi_a - lo_a
        wb = hi_b - lo_b
        qa = lo_a + (iota1 * wa) // K
        qb = lo_b + (iota1 * wb) // K
        idx_v[pl.ds(0, K)] = jnp.minimum(qa, M_ELEM - 1)
        idx_v[pl.ds(K, K)] = jnp.minimum(qb, M_ELEM - 1)
        pltpu.async_copy(e_hbm.at[idx_v], probe_v, sem0).wait()
        va = jnp.where(qa >= M_ELEM, big, probe_v[pl.ds(0, K)])
        vb = jnp.where(qb >= M_ELEM, big, probe_v[pl.ds(K, K)])
        ca = jnp.sum((va < tgt_a).astype(jnp.int32))
        cb = jnp.sum((vb < tgt_b).astype(jnp.int32))
        new_hi_a = lo_a + ((ca + 1) * wa) // K
        new_lo_a = lo_a + jnp.where(ca > 0, (ca * wa) // K + 1, 0)
        new_hi_b = lo_b + ((cb + 1) * wb) // K
        new_lo_b = lo_b + jnp.where(cb > 0, (cb * wb) // K + 1, 0)
        return new_lo_a, new_hi_a, new_lo_b, new_hi_b

    start, _, end, _ = lax.fori_loop(
        0, 7, bs_body, (jnp.int32(0), jnp.int32(M_ELEM),
                        jnp.int32(0), jnp.int32(M_ELEM)))

    def zero_body(i, c):
        yloc[pl.ds(i * K, K)] = jnp.zeros((K,), jnp.float32)
        return c

    lax.fori_loop(0, YLOC // K, zero_body, 0)

    def flush_val(top):
        return jnp.sum(jnp.where(top == _negv(), 0.0, top) * wk)

    start8 = start // 8 * 8
    n_chunks = (end - start8 + CHUNK - 1) // CHUNK

    sems = (sem0, sem1)

    def _chunk_start(k):
        return jnp.minimum(start8 + k * CHUNK, M_ELEM - CHUNK)

    def _dma_pair(k, slot):
        cs = _chunk_start(k)
        off = slot * CHUNK
        a = pltpu.make_async_copy(
            e_hbm.at[pl.ds(cs, CHUNK)], ids_v.at[pl.ds(off, CHUNK)], sems[slot])
        b = pltpu.make_async_copy(
            yv_hbm.at[pl.ds(cs, CHUNK)], vals_v.at[pl.ds(off, CHUNK)], sems[slot])
        return a, b

    def _start_dma(k, slot):
        a, b = _dma_pair(k, slot)
        a.start()
        b.start()

    def chunk_body(k, slot, carry):
        a, b = _dma_pair(k, slot)
        a.wait()
        b.wait()
        nominal = start8 + k * CHUNK
        cstart = _chunk_start(k)
        boff = slot * CHUNK
        lo = jnp.maximum(start, nominal)

        def one_vec(v, ids_raw, vals_raw, ps, cur, top):
            base = cstart + v * K
            all_valid = (base >= lo) & (base + K <= end)
            uniform = (ids_raw[0] == cur) & (ids_raw[K - 1] == cur)

            def fast(cur, top):
                m = jnp.maximum(top, jnp.flip(ps, 0))
                ts, _ = plsc.sort_key_val(m, m, descending=True)
                return cur, ts

            def slow(cur, top):
                gidx = base + iota
                valid = (gidx >= lo) & (gidx < end)
                vals_m = jnp.where(valid, vals_raw, -jnp.inf)
                ids_fill = jnp.where(valid, ids_raw, cur).astype(jnp.float32)
                srt, _ = plsc.sort_key_val(ids_fill, ids_fill, descending=True)
                mx = srt[0].astype(jnp.int32)
                ids_t = jnp.where(valid, ids_raw, jnp.where(gidx < lo, cur, mx))
                tmp_ids[pl.ds(0, K)] = ids_t

                def wcond(st):
                    return st[0] < K

                def wbody(st):
                    pos, cur, top = st
                    seg = tmp_ids[pl.ds(pos, K)][0]
                    mask = ids_t == seg
                    n = jnp.sum(mask.astype(jnp.int32))
                    piece = jnp.where(mask, vals_m, -jnp.inf)
                    do_flush = seg != cur
                    contrib = flush_val(top)
                    plsc.store_scatter(
                        yloc,
                        (jnp.full((K,), cur - s0, jnp.int32),),
                        jnp.full((K,), contrib, jnp.float32),
                        mask=lane0 & do_flush,
                    )
                    top = jnp.where(do_flush, _negv(), top)
                    cur = jnp.where(do_flush, seg, cur)
                    top = _merge_top16(top, piece)
                    return pos + n, cur, top

                _, cur, top = lax.while_loop(
                    wcond, wbody, (jnp.int32(0), cur, top))
                return cur, top

            return lax.cond(all_valid & uniform, fast, slow, cur, top)

        def pair_body(u, c2):
            cur, top = c2
            b0 = boff + (2 * u) * K
            ids0 = ids_v[pl.ds(b0, K)]
            vals0 = vals_v[pl.ds(b0, K)]
            ids1 = ids_v[pl.ds(b0 + K, K)]
            vals1 = vals_v[pl.ds(b0 + K, K)]
            ps0, _ = plsc.sort_key_val(vals0, vals0, descending=True)
            ps1, _ = plsc.sort_key_val(vals1, vals1, descending=True)
            base0 = cstart + (2 * u) * K
            pair_valid = (base0 >= lo) & (base0 + 2 * K <= end)
            pair_fast = pair_valid & (ids0[0] == cur) & (ids1[K - 1] == cur)

            def fastpair(cur, top):
                m01 = jnp.maximum(ps0, jnp.flip(ps1, 0))
                ps01, _ = plsc.sort_key_val(m01, m01, descending=True)
                m = jnp.maximum(top, jnp.flip(ps01, 0))
                ts, _ = plsc.sort_key_val(m, m, descending=True)
                return cur, ts

            def pervec(cur, top):
                cur, top = one_vec(2 * u, ids0, vals0, ps0, cur, top)
                cur, top = one_vec(2 * u + 1, ids1, vals1, ps1, cur, top)
                return cur, top

            return lax.cond(pair_fast, fastpair, pervec, cur, top)

        return lax.fori_loop(0, CHUNK // (2 * K), pair_body, carry)

    @pl.when(n_chunks > 0)
    def _():
        _start_dma(0, 0)

    def pair_body(p, carry):
        for b in range(2):
            k = 2 * p + b

            def go(c, k=k, b=b):
                @pl.when(k + 1 < n_chunks)
                def _():
                    _start_dma(k + 1, 1 - b)

                return chunk_body(k, b, c)

            carry = lax.cond(k < n_chunks, go, lambda c: c, carry)
        return carry

    n_pairs = (n_chunks + 1) // 2
    cur, top = lax.fori_loop(0, n_pairs, pair_body, (s0, _negv()))
    plsc.store_scatter(
        yloc,
        (jnp.full((K,), cur - s0, jnp.int32),),
        jnp.full((K,), flush_val(top), jnp.float32),
        mask=lane0,
    )

    @pl.when(wid != NW - 1)
    def _():
        pltpu.sync_copy(yloc.at[pl.ds(0, SEG_PER_W)],
                        out_hbm.at[pl.ds(s0, SEG_PER_W)])

    @pl.when(wid == NW - 1)
    def _():
        pltpu.sync_copy(yloc.at[pl.ds(0, SEG_LAST)],
                        out_hbm.at[pl.ds(s0, SEG_LAST)])


@functools.lru_cache(maxsize=1)
def _build_sc_kernel():
    mesh = plsc.VectorSubcoreMesh(core_axis_name="c", subcore_axis_name="s")
    return pl.kernel(
        _sort_pool_body,
        out_type=jax.ShapeDtypeStruct((N_SEG,), jnp.float32),
        mesh=mesh,
        compiler_params=pltpu.CompilerParams(needs_layout_passes=False),
        scratch_types=[
            pltpu.VMEM((2 * CHUNK,), jnp.int32),
            pltpu.VMEM((2 * CHUNK,), jnp.float32),
            pltpu.VMEM((YLOC,), jnp.float32),
            pltpu.VMEM((K,), jnp.float32),
            pltpu.VMEM((2 * K,), jnp.int32),
            pltpu.VMEM((2 * K,), jnp.int32),
            pltpu.VMEM((2 * K,), jnp.int32),
            pltpu.SemaphoreType.DMA,
            pltpu.SemaphoreType.DMA,
        ],
    )


def kernel(e_map, v_count, Y, W):
    del v_count
    yf = jnp.squeeze(Y, -1)
    return _build_sc_kernel()(e_map, yf, W.astype(jnp.float32))


# speedup vs baseline: 1.0204x; 1.0204x over previous
import functools

import jax
import jax.numpy as jnp
from jax import lax
from jax.experimental import pallas as pl
from jax.experimental.pallas import tpu as pltpu
from jax.experimental.pallas import tpu_sc as plsc

N_SEG = 100000
M_ELEM = 6400000
K = 16
NW = 32
SEG_PER_W = 3128
SEG_LAST = N_SEG - (NW - 1) * SEG_PER_W
YLOC = 3136
CHUNK = 16384
NB = 48

def _merge_top16(top, piece):
    ps, _ = plsc.sort_key_val(piece, piece, descending=True)
    m = jnp.maximum(top, jnp.flip(ps, 0))
    ts, _ = plsc.sort_key_val(m, m, descending=True)
    return ts


def _negv():
    return jnp.full((K,), -jnp.inf, dtype=jnp.float32)


def _sort_pool_body(e_hbm, yv_hbm, w_hbm, out_hbm,
                    ids_v, vals_v, yloc, wk_v, tmp_ids, idx_v, probe_v,
                    sem0, sem1):
    wid = lax.axis_index("c") * 16 + lax.axis_index("s")
    pltpu.sync_copy(w_hbm, wk_v)
    iota = lax.iota(jnp.int32, K)
    lane0 = iota == 0

    w_raw = wk_v[...]
    wsrt, _ = plsc.sort_key_val(w_raw, w_raw, descending=True)
    w_exp = jnp.exp(w_raw - wsrt[0])
    wk = w_exp / jnp.sum(w_exp)

    s0 = wid * SEG_PER_W
    s1m1 = jnp.minimum(s0 + SEG_PER_W, N_SEG) - 1

    big = jnp.int32(2 ** 30)
    iota1 = iota + 1
    tgt_a = s0
    tgt_b = s0 + SEG_PER_W

    def bs_body(i, st):
        lo_a, hi_a, lo_b, hi_b = st
        wa = hi_a - lo_a
        wb = hi_b - lo_b
        qa = lo_a + (iota1 * wa) // K
        qb = lo_b + (iota1 * wb) // K
        idx_v[pl.ds(0, K)] = jnp.minimum(qa, M_ELEM - 1)
        idx_v[pl.ds(K, K)] = jnp.minimum(qb, M_ELEM - 1)
        pltpu.async_copy(e_hbm.at[idx_v], probe_v, sem0).wait()
        va = jnp.where(qa >= M_ELEM, big, probe_v[pl.ds(0, K)])
        vb = jnp.where(qb >= M_ELEM, big, probe_v[pl.ds(K, K)])
        ca = jnp.sum((va < tgt_a).astype(jnp.int32))
        cb = jnp.sum((vb < tgt_b).astype(jnp.int32))
        new_hi_a = lo_a + ((ca + 1) * wa) // K
        new_lo_a = lo_a + jnp.where(ca > 0, (ca * wa) // K + 1, 0)
        new_hi_b = lo_b + ((cb + 1) * wb) // K
        new_lo_b = lo_b + jnp.where(cb > 0, (cb * wb) // K + 1, 0)
        return new_lo_a, new_hi_a, new_lo_b, new_hi_b

    start, _, end, _ = lax.fori_loop(
        0, 7, bs_body, (jnp.int32(0), jnp.int32(M_ELEM),
                        jnp.int32(0), jnp.int32(M_ELEM)))

    def zero_body(i, c):
        yloc[pl.ds(i * K, K)] = jnp.zeros((K,), jnp.float32)
        return c

    lax.fori_loop(0, YLOC // K, zero_body, 0)

    def flush_val(top):
        return jnp.sum(jnp.where(top == _negv(), 0.0, top) * wk)

    start8 = start // 8 * 8
    n_chunks = (end - start8 + CHUNK - 1) // CHUNK

    sems = (sem0, sem1)

    def _chunk_start(k):
        return jnp.minimum(start8 + k * CHUNK, M_ELEM - CHUNK)

    def _dma_pair(k, slot):
        cs = _chunk_start(k)
        off = slot * CHUNK
        a = pltpu.make_async_copy(
            e_hbm.at[pl.ds(cs, CHUNK)], ids_v.at[pl.ds(off, CHUNK)], sems[slot])
        b = pltpu.make_async_copy(
            yv_hbm.at[pl.ds(cs, CHUNK)], vals_v.at[pl.ds(off, CHUNK)], sems[slot])
        return a, b

    def _start_dma(k, slot):
        a, b = _dma_pair(k, slot)
        a.start()
        b.start()

    def chunk_body(k, slot, carry):
        a, b = _dma_pair(k, slot)
        a.wait()
        b.wait()
        nominal = start8 + k * CHUNK
        cstart = _chunk_start(k)
        boff = slot * CHUNK
        lo = jnp.maximum(start, nominal)

        def one_vec(v, ids_raw, vals_raw, ps_asc, cur, top):
            base = cstart + v * K
            all_valid = (base >= lo) & (base + K <= end)
            uniform = (ids_raw[0] == cur) & (ids_raw[K - 1] == cur)

            def fast(cur, top):
                m = jnp.maximum(top, ps_asc)
                ts, _ = plsc.sort_key_val(m, m, descending=True)
                return cur, ts

            def slow(cur, top):
                gidx = base + iota
                valid = (gidx >= lo) & (gidx < end)
                vals_m = jnp.where(valid, vals_raw, -jnp.inf)
                ids_fill = jnp.where(valid, ids_raw, cur).astype(jnp.float32)
                srt, _ = plsc.sort_key_val(ids_fill, ids_fill, descending=True)
                mx = srt[0].astype(jnp.int32)
                ids_t = jnp.where(valid, ids_raw, jnp.where(gidx < lo, cur, mx))
                tmp_ids[pl.ds(0, K)] = ids_t

                def wcond(st):
                    return st[0] < K

                def wbody(st):
                    pos, cur, top = st
                    seg = tmp_ids[pl.ds(pos, K)][0]
                    mask = ids_t == seg
                    n = jnp.sum(mask.astype(jnp.int32))
                    piece = jnp.where(mask, vals_m, -jnp.inf)
                    do_flush = seg != cur
                    contrib = flush_val(top)
                    plsc.store_scatter(
                        yloc,
                        (jnp.full((K,), cur - s0, jnp.int32),),
                        jnp.full((K,), contrib, jnp.float32),
                        mask=lane0 & do_flush,
                    )
                    top = jnp.where(do_flush, _negv(), top)
                    cur = jnp.where(do_flush, seg, cur)
                    top = _merge_top16(top, piece)
                    return pos + n, cur, top

                _, cur, top = lax.while_loop(
                    wcond, wbody, (jnp.int32(0), cur, top))
                return cur, top

            return lax.cond(all_valid & uniform, fast, slow, cur, top)

        def pair_body(u, c2):
            cur, top = c2
            b0 = boff + (2 * u) * K
            ids0 = ids_v[pl.ds(b0, K)]
            vals0 = vals_v[pl.ds(b0, K)]
            ids1 = ids_v[pl.ds(b0 + K, K)]
            vals1 = vals_v[pl.ds(b0 + K, K)]
            ps0, _ = plsc.sort_key_val(vals0, vals0, descending=True)
            ps1a, _ = plsc.sort_key_val(vals1, vals1, descending=False)
            base0 = cstart + (2 * u) * K
            pair_valid = (base0 >= lo) & (base0 + 2 * K <= end)
            pair_fast = pair_valid & (ids0[0] == cur) & (ids1[K - 1] == cur)

            def fastpair(cur, top):
                m01 = jnp.maximum(ps0, ps1a)
                ps01a, _ = plsc.sort_key_val(m01, m01, descending=False)
                m = jnp.maximum(top, ps01a)
                ts, _ = plsc.sort_key_val(m, m, descending=True)
                return cur, ts

            def pervec(cur, top):
                cur, top = one_vec(2 * u, ids0, vals0, jnp.flip(ps0, 0),
                                   cur, top)
                cur, top = one_vec(2 * u + 1, ids1, vals1, ps1a, cur, top)
                return cur, top

            return lax.cond(pair_fast, fastpair, pervec, cur, top)

        return lax.fori_loop(0, CHUNK // (2 * K), pair_body, carry)

    @pl.when(n_chunks > 0)
    def _():
        _start_dma(0, 0)

    def pair_body(p, carry):
        for b in range(2):
            k = 2 * p + b

            def go(c, k=k, b=b):
                @pl.when(k + 1 < n_chunks)
                def _():
                    _start_dma(k + 1, 1 - b)

                return chunk_body(k, b, c)

            carry = lax.cond(k < n_chunks, go, lambda c: c, carry)
        return carry

    n_pairs = (n_chunks + 1) // 2
    cur, top = lax.fori_loop(0, n_pairs, pair_body, (s0, _negv()))
    plsc.store_scatter(
        yloc,
        (jnp.full((K,), cur - s0, jnp.int32),),
        jnp.full((K,), flush_val(top), jnp.float32),
        mask=lane0,
    )

    @pl.when(wid != NW - 1)
    def _():
        pltpu.sync_copy(yloc.at[pl.ds(0, SEG_PER_W)],
                        out_hbm.at[pl.ds(s0, SEG_PER_W)])

    @pl.when(wid == NW - 1)
    def _():
        pltpu.sync_copy(yloc.at[pl.ds(0, SEG_LAST)],
                        out_hbm.at[pl.ds(s0, SEG_LAST)])


@functools.lru_cache(maxsize=1)
def _build_sc_kernel():
    mesh = plsc.VectorSubcoreMesh(core_axis_name="c", subcore_axis_name="s")
    return pl.kernel(
        _sort_pool_body,
        out_type=jax.ShapeDtypeStruct((N_SEG,), jnp.float32),
        mesh=mesh,
        compiler_params=pltpu.CompilerParams(needs_layout_passes=False),
        scratch_types=[
            pltpu.VMEM((2 * CHUNK,), jnp.int32),
            pltpu.VMEM((2 * CHUNK,), jnp.float32),
            pltpu.VMEM((YLOC,), jnp.float32),
            pltpu.VMEM((K,), jnp.float32),
            pltpu.VMEM((2 * K,), jnp.int32),
            pltpu.VMEM((2 * K,), jnp.int32),
            pltpu.VMEM((2 * K,), jnp.int32),
            pltpu.SemaphoreType.DMA,
            pltpu.SemaphoreType.DMA,
        ],
    )


def kernel(e_map, v_count, Y, W):
    del v_count
    yf = jnp.squeeze(Y, -1)
    return _build_sc_kernel()(e_map, yf, W.astype(jnp.float32))


# speedup vs baseline: 1.0417x; 1.0209x over previous
import functools

import jax
import jax.numpy as jnp
from jax import lax
from jax.experimental import pallas as pl
from jax.experimental.pallas import tpu as pltpu
from jax.experimental.pallas import tpu_sc as plsc

N_SEG = 100000
M_ELEM = 6400000
K = 16
NW = 32
SEG_PER_W = 3128
SEG_LAST = N_SEG - (NW - 1) * SEG_PER_W
YLOC = 3136
CHUNK = 16384
NB = 48

def _merge_top16(top, piece):
    ps, _ = plsc.sort_key_val(piece, piece, descending=True)
    m = jnp.maximum(top, jnp.flip(ps, 0))
    ts, _ = plsc.sort_key_val(m, m, descending=True)
    return ts


def _negv():
    return jnp.full((K,), -jnp.inf, dtype=jnp.float32)


def _sort_pool_body(e_hbm, yv_hbm, w_hbm, out_hbm,
                    ids_v, vals_v, yloc, wk_v, tmp_ids, idx_v, probe_v,
                    sem0, sem1):
    wid = lax.axis_index("c") * 16 + lax.axis_index("s")
    pltpu.sync_copy(w_hbm, wk_v)
    iota = lax.iota(jnp.int32, K)
    lane0 = iota == 0

    w_raw = wk_v[...]
    wsrt, _ = plsc.sort_key_val(w_raw, w_raw, descending=True)
    w_exp = jnp.exp(w_raw - wsrt[0])
    wk = w_exp / jnp.sum(w_exp)

    s0 = wid * SEG_PER_W
    s1m1 = jnp.minimum(s0 + SEG_PER_W, N_SEG) - 1

    big = jnp.int32(2 ** 30)
    iota1 = iota + 1
    tgt_a = s0
    tgt_b = s0 + SEG_PER_W

    def bs_body(i, st):
        lo_a, hi_a, lo_b, hi_b = st
        wa = hi_a - lo_a
        wb = hi_b - lo_b
        qa = lo_a + (iota1 * wa) // K
        qb = lo_b + (iota1 * wb) // K
        idx_v[pl.ds(0, K)] = jnp.minimum(qa, M_ELEM - 1)
        idx_v[pl.ds(K, K)] = jnp.minimum(qb, M_ELEM - 1)
        pltpu.async_copy(e_hbm.at[idx_v], probe_v, sem0).wait()
        va = jnp.where(qa >= M_ELEM, big, probe_v[pl.ds(0, K)])
        vb = jnp.where(qb >= M_ELEM, big, probe_v[pl.ds(K, K)])
        ca = jnp.sum((va < tgt_a).astype(jnp.int32))
        cb = jnp.sum((vb < tgt_b).astype(jnp.int32))
        new_hi_a = lo_a + ((ca + 1) * wa) // K
        new_lo_a = lo_a + jnp.where(ca > 0, (ca * wa) // K + 1, 0)
        new_hi_b = lo_b + ((cb + 1) * wb) // K
        new_lo_b = lo_b + jnp.where(cb > 0, (cb * wb) // K + 1, 0)
        return new_lo_a, new_hi_a, new_lo_b, new_hi_b

    start, _, end, _ = lax.fori_loop(
        0, 7, bs_body, (jnp.int32(0), jnp.int32(M_ELEM),
                        jnp.int32(0), jnp.int32(M_ELEM)))

    def zero_body(i, c):
        yloc[pl.ds(i * K, K)] = jnp.zeros((K,), jnp.float32)
        return c

    lax.fori_loop(0, YLOC // K, zero_body, 0)

    def flush_val(top):
        return jnp.sum(jnp.where(top == _negv(), 0.0, top) * wk)

    start8 = start // 8 * 8
    n_chunks = (end - start8 + CHUNK - 1) // CHUNK

    sems = (sem0, sem1)

    def _chunk_start(k):
        return jnp.minimum(start8 + k * CHUNK, M_ELEM - CHUNK)

    def _dma_pair(k, slot):
        cs = _chunk_start(k)
        off = slot * CHUNK
        a = pltpu.make_async_copy(
            e_hbm.at[pl.ds(cs, CHUNK)], ids_v.at[pl.ds(off, CHUNK)], sems[slot])
        b = pltpu.make_async_copy(
            yv_hbm.at[pl.ds(cs, CHUNK)], vals_v.at[pl.ds(off, CHUNK)], sems[slot])
        return a, b

    def _start_dma(k, slot):
        a, b = _dma_pair(k, slot)
        a.start()
        b.start()

    def chunk_body(k, slot, carry):
        a, b = _dma_pair(k, slot)
        a.wait()
        b.wait()
        nominal = start8 + k * CHUNK
        cstart = _chunk_start(k)
        boff = slot * CHUNK
        lo = jnp.maximum(start, nominal)

        def one_vec(v, ids_raw, vals_raw, ps_asc, cur, top):
            base = cstart + v * K
            all_valid = (base >= lo) & (base + K <= end)
            uniform = (ids_raw[0] == cur) & (ids_raw[K - 1] == cur)

            def fast(cur, top):
                m = jnp.maximum(top, ps_asc)
                ts, _ = plsc.sort_key_val(m, m, descending=True)
                return cur, ts

            def slow(cur, top):
                gidx = base + iota
                valid = (gidx >= lo) & (gidx < end)
                vals_m = jnp.where(valid, vals_raw, -jnp.inf)
                ids_fill = jnp.where(valid, ids_raw, cur).astype(jnp.float32)
                srt, _ = plsc.sort_key_val(ids_fill, ids_fill, descending=True)
                mx = srt[0].astype(jnp.int32)
                ids_t = jnp.where(valid, ids_raw, jnp.where(gidx < lo, cur, mx))
                tmp_ids[pl.ds(0, K)] = ids_t

                def wcond(st):
                    return st[0] < K

                def wbody(st):
                    pos, cur, top = st
                    seg = tmp_ids[pl.ds(pos, K)][0]
                    mask = ids_t == seg
                    n = jnp.sum(mask.astype(jnp.int32))
                    piece = jnp.where(mask, vals_m, -jnp.inf)
                    do_flush = seg != cur
                    contrib = flush_val(top)
                    plsc.store_scatter(
                        yloc,
                        (jnp.full((K,), cur - s0, jnp.int32),),
                        jnp.full((K,), contrib, jnp.float32),
                        mask=lane0 & do_flush,
                    )
                    top = jnp.where(do_flush, _negv(), top)
                    cur = jnp.where(do_flush, seg, cur)
                    top = _merge_top16(top, piece)
                    return pos + n, cur, top

                _, cur, top = lax.while_loop(
                    wcond, wbody, (jnp.int32(0), cur, top))
                return cur, top

            return lax.cond(all_valid & uniform, fast, slow, cur, top)

        def pair_body(u, c2):
            cur, top = c2
            b0 = boff + (2 * u) * K
            ids0 = ids_v[pl.ds(b0, K)]
            vals0 = vals_v[pl.ds(b0, K)]
            ids1 = ids_v[pl.ds(b0 + K, K)]
            vals1 = vals_v[pl.ds(b0 + K, K)]
            ps0, _ = plsc.sort_key_val(vals0, vals0, descending=True)
            ps1a, _ = plsc.sort_key_val(vals1, vals1, descending=False)
            base0 = cstart + (2 * u) * K
            pair_valid = (base0 >= lo) & (base0 + 2 * K <= end)
            pair_fast = pair_valid & (ids0[0] == cur) & (ids1[K - 1] == cur)

            def fastpair(cur, top):
                m01 = jnp.maximum(ps0, ps1a)
                ps01a, _ = plsc.sort_key_val(m01, m01, descending=False)
                m = jnp.maximum(top, ps01a)
                ts, _ = plsc.sort_key_val(m, m, descending=True)
                return cur, ts

            def pervec(cur, top):
                cur, top = one_vec(2 * u, ids0, vals0, jnp.flip(ps0, 0),
                                   cur, top)
                cur, top = one_vec(2 * u + 1, ids1, vals1, ps1a, cur, top)
                return cur, top

            return lax.cond(pair_fast, fastpair, pervec, cur, top)

        def pair2_body(w, c2):
            c2 = pair_body(2 * w, c2)
            return pair_body(2 * w + 1, c2)

        return lax.fori_loop(0, CHUNK // (4 * K), pair2_body, carry)

    @pl.when(n_chunks > 0)
    def _():
        _start_dma(0, 0)

    def pair_body(p, carry):
        for b in range(2):
            k = 2 * p + b

            def go(c, k=k, b=b):
                @pl.when(k + 1 < n_chunks)
                def _():
                    _start_dma(k + 1, 1 - b)

                return chunk_body(k, b, c)

            carry = lax.cond(k < n_chunks, go, lambda c: c, carry)
        return carry

    n_pairs = (n_chunks + 1) // 2
    cur, top = lax.fori_loop(0, n_pairs, pair_body, (s0, _negv()))
    plsc.store_scatter(
        yloc,
        (jnp.full((K,), cur - s0, jnp.int32),),
        jnp.full((K,), flush_val(top), jnp.float32),
        mask=lane0,
    )

    @pl.when(wid != NW - 1)
    def _():
        pltpu.sync_copy(yloc.at[pl.ds(0, SEG_PER_W)],
                        out_hbm.at[pl.ds(s0, SEG_PER_W)])

    @pl.when(wid == NW - 1)
    def _():
        pltpu.sync_copy(yloc.at[pl.ds(0, SEG_LAST)],
                        out_hbm.at[pl.ds(s0, SEG_LAST)])


@functools.lru_cache(maxsize=1)
def _build_sc_kernel():
    mesh = plsc.VectorSubcoreMesh(core_axis_name="c", subcore_axis_name="s")
    return pl.kernel(
        _sort_pool_body,
        out_type=jax.ShapeDtypeStruct((N_SEG,), jnp.float32),
        mesh=mesh,
        compiler_params=pltpu.CompilerParams(needs_layout_passes=False),
        scratch_types=[
            pltpu.VMEM((2 * CHUNK,), jnp.int32),
            pltpu.VMEM((2 * CHUNK,), jnp.float32),
            pltpu.VMEM((YLOC,), jnp.float32),
            pltpu.VMEM((K,), jnp.float32),
            pltpu.VMEM((2 * K,), jnp.int32),
            pltpu.VMEM((2 * K,), jnp.int32),
            pltpu.VMEM((2 * K,), jnp.int32),
            pltpu.SemaphoreType.DMA,
            pltpu.SemaphoreType.DMA,
        ],
    )


def kernel(e_map, v_count, Y, W):
    del v_count
    yf = jnp.squeeze(Y, -1)
    return _build_sc_kernel()(e_map, yf, W.astype(jnp.float32))
